# Initial kernel scaffold; baseline (speedup 1.0000x reference)
import jax
import jax.numpy as jnp
from jax.experimental import pallas as pl


def kernel(h, W):
    raise NotImplementedError("write your pallas kernel here")


# speedup vs baseline: 1.0784x; 1.0784x over previous
import functools

import jax
import jax.numpy as jnp
from jax import lax
from jax.experimental import pallas as pl
from jax.experimental.pallas import tpu as pltpu

_D_MODEL = 4096
_N_EXPERTS = 64
_TOP_K = 8
_N_TOKENS = 32768
_BT = 1024


def _router_body(h_ref, w_ref, idx_ref, wgt_ref, logits_ref):
    logits = lax.dot_general(
        h_ref[...], w_ref[...],
        dimension_numbers=(((1,), (1,)), ((), ())),
        preferred_element_type=jnp.float32,
    )
    logits_ref[...] = logits

    e_iota = lax.broadcasted_iota(jnp.int32, (_BT, _N_EXPERTS), 1)
    x = logits
    vals = []
    idxs = []
    for _ in range(_TOP_K):
        m = jnp.max(x, axis=1, keepdims=True)
        am = jnp.min(
            jnp.where(x == m, e_iota, _N_EXPERTS), axis=1, keepdims=True
        )
        vals.append(m)
        idxs.append(am)
        x = jnp.where(e_iota == am, -jnp.inf, x)

    topv = jnp.concatenate(vals, axis=1)
    topi = jnp.concatenate(idxs, axis=1)

    e = jnp.exp(topv - topv[:, :1])
    wgt = e / jnp.sum(e, axis=1, keepdims=True)

    idx_ref[...] = topi
    wgt_ref[...] = wgt


@jax.jit
def kernel(h, W):
    n_tokens = h.shape[0]
    grid = (n_tokens // _BT,)
    out_shapes = (
        jax.ShapeDtypeStruct((n_tokens, _TOP_K), jnp.int32),
        jax.ShapeDtypeStruct((n_tokens, _TOP_K), jnp.float32),
        jax.ShapeDtypeStruct((n_tokens, _N_EXPERTS), jnp.float32),
    )
    topi, wgt, logits = pl.pallas_call(
        _router_body,
        grid=grid,
        in_specs=[
            pl.BlockSpec((_BT, _D_MODEL), lambda i: (i, 0)),
            pl.BlockSpec((_N_EXPERTS, _D_MODEL), lambda i: (0, 0)),
        ],
        out_specs=(
            pl.BlockSpec((_BT, _TOP_K), lambda i: (i, 0)),
            pl.BlockSpec((_BT, _TOP_K), lambda i: (i, 0)),
            pl.BlockSpec((_BT, _N_EXPERTS), lambda i: (i, 0)),
        ),
        out_shape=out_shapes,
    )(h, W)
    return (topi, wgt, logits)
